# baseline (speedup 1.0000x reference)
import functools
import math

import jax
import jax.numpy as jnp
from jax import lax
from jax.experimental import pallas as pl
from jax.experimental.pallas import tpu as pltpu
from jax.experimental.pallas import tpu_sc as plsc


def _uv_body(W, H, ld_ref, i00_ref, i01_ref, i10_ref, i11_ref, wx_ref, wy_ref):
    x = ld_ref[0, :]
    y = ld_ref[1, :]
    z = ld_ref[2, :]
    norm = jnp.sqrt(x * x + y * y + z * z)
    yn = y / norm
    yc = jnp.clip(yn, -1.0 + 1e-06, 1.0 - 1e-06)
    theta = jnp.arctan2(jnp.sqrt((1.0 + yc) * (1.0 - yc)), yc)
    phi = jnp.arctan2(x, z)
    u = phi / (2.0 * math.pi) + 0.5
    u = u - jnp.floor(u)
    v = theta / math.pi
    xf = jnp.clip(u * W, 0.0, W - 1.0)
    yf = jnp.clip(v * H, 0.0, H - 1.0)
    x0f = jnp.floor(xf)
    y0f = jnp.floor(yf)
    wx_ref[...] = xf - x0f
    wy_ref[...] = yf - y0f
    x0 = x0f.astype(jnp.int32)
    y0 = y0f.astype(jnp.int32)
    x1 = jnp.minimum(x0 + 1, int(W) - 1)
    y1 = jnp.minimum(y0 + 1, int(H) - 1)
    r0 = y0 * int(W)
    r1 = y1 * int(W)
    i00_ref[...] = r0 + x0
    i01_ref[...] = r0 + x1
    i10_ref[...] = r1 + x0
    i11_ref[...] = r1 + x1


def _uv_kernel(ldT, H, W, TB=8192):
    B = ldT.shape[1]
    G = B // TB
    iout = jax.ShapeDtypeStruct((B,), jnp.int32)
    fout = jax.ShapeDtypeStruct((B,), jnp.float32)
    ospec = pl.BlockSpec((TB,), lambda i: (i,))
    outs = pl.pallas_call(
        functools.partial(_uv_body, float(W), float(H)),
        grid=(G,),
        in_specs=[pl.BlockSpec((3, TB), lambda i: (0, i))],
        out_specs=[ospec] * 6,
        out_shape=[iout, iout, iout, iout, fout, fout],
    )(ldT)
    return outs


_LANES = 16


def _sc_gather_combine(env_flat, i00, i01, i10, i11, wx, wy, HW, C=1024):
    B = i00.shape[0]
    info = plsc.get_sparse_core_info()
    NC, NS = info.num_cores, info.num_subcores
    NW = NC * NS
    RW = B // NW
    NCHUNK = RW // C
    GROUPS = C // _LANES

    mesh = plsc.VectorSubcoreMesh(core_axis_name="c", subcore_axis_name="s")

    @functools.partial(
        pl.kernel,
        out_type=jax.ShapeDtypeStruct((B * 3,), jnp.float32),
        mesh=mesh,
        scratch_types=[
            pltpu.VMEM((4, C), jnp.int32),
            pltpu.VMEM((12, C), jnp.int32),
            pltpu.VMEM((12, C), jnp.float32),
            pltpu.VMEM((C,), jnp.float32),
            pltpu.VMEM((C,), jnp.float32),
            pltpu.VMEM((3 * C,), jnp.float32),
            pltpu.SemaphoreType.DMA,
        ],
        compiler_params=pltpu.CompilerParams(
            needs_layout_passes=False, use_tc_tiling_on_sc=False),
    )
    def body(env_hbm, i00_hbm, i01_hbm, i10_hbm, i11_hbm, wx_hbm, wy_hbm,
             le_hbm, ic_v, idx_v, tex_v, wx_v, wy_v, out_v, sem):
        wid = lax.axis_index("s") * NC + lax.axis_index("c")
        base = wid * RW

        def chunk(t, carry):
            b0 = base + t * C
            pltpu.sync_copy(i00_hbm.at[pl.ds(b0, C)], ic_v.at[0])
            pltpu.sync_copy(i01_hbm.at[pl.ds(b0, C)], ic_v.at[1])
            pltpu.sync_copy(i10_hbm.at[pl.ds(b0, C)], ic_v.at[2])
            pltpu.sync_copy(i11_hbm.at[pl.ds(b0, C)], ic_v.at[3])
            pltpu.sync_copy(wx_hbm.at[pl.ds(b0, C)], wx_v)
            pltpu.sync_copy(wy_hbm.at[pl.ds(b0, C)], wy_v)

            def build(g, gcarry):
                s = pl.ds(g * _LANES, _LANES)
                for corner in range(4):
                    raw = ic_v[corner, s]
                    for ch in range(3):
                        idx_v[corner * 3 + ch, s] = raw + (ch * HW)
                return gcarry

            lax.fori_loop(0, GROUPS, build, 0)

            cps = [
                pltpu.async_copy(env_hbm.at[idx_v.at[j]], tex_v.at[j], sem)
                for j in range(12)
            ]
            for cp in cps:
                cp.wait()

            iota = lax.iota(jnp.int32, _LANES)

            def group(g, gcarry):
                s = pl.ds(g * _LANES, _LANES)
                wxv = wx_v[s]
                wyv = wy_v[s]
                pos = (g * _LANES + iota) * 3
                for ch in range(3):
                    a = tex_v[0 + ch, s]
                    b = tex_v[3 + ch, s]
                    c = tex_v[6 + ch, s]
                    d = tex_v[9 + ch, s]
                    top = a + wxv * (b - a)
                    bot = c + wxv * (d - c)
                    le = (top + wyv * (bot - top)) * (1.0 / 256.0)
                    plsc.store_scatter(out_v, [pos + ch], le)
                return gcarry

            lax.fori_loop(0, GROUPS, group, 0)
            pltpu.sync_copy(out_v, le_hbm.at[pl.ds(b0 * 3, C * 3)])
            return carry

        lax.fori_loop(0, NCHUNK, chunk, 0)

    return body(env_flat, i00, i01, i10, i11, wx, wy).reshape(B, 3)


def kernel(position, light_dir, envmap):
    B = light_dir.shape[0]
    H, W = envmap.shape[1], envmap.shape[2]
    ldT = light_dir.T
    i00, i01, i10, i11, wx, wy = _uv_kernel(ldT, H, W)
    env_flat = envmap.reshape(3 * H * W)
    le = _sc_gather_combine(env_flat, i00, i01, i10, i11, wx, wy, H * W)
    pdf = jnp.full((B, 1), 1.0 / (4 * math.pi), dtype=jnp.float32)
    valid = jnp.ones((B, 1), dtype=bool)
    return (le, pdf, valid)


# speedup vs baseline: 2.3644x; 2.3644x over previous
import functools
import math

import jax
import jax.numpy as jnp
from jax import lax
from jax.experimental import pallas as pl
from jax.experimental.pallas import tpu as pltpu
from jax.experimental.pallas import tpu_sc as plsc


def _uv_body(W, H, ld_ref, i00_ref, i01_ref, i10_ref, i11_ref, wx_ref, wy_ref):
    x = ld_ref[0, :]
    y = ld_ref[1, :]
    z = ld_ref[2, :]
    norm = jnp.sqrt(x * x + y * y + z * z)
    yn = y / norm
    yc = jnp.clip(yn, -1.0 + 1e-06, 1.0 - 1e-06)
    theta = jnp.arctan2(jnp.sqrt((1.0 + yc) * (1.0 - yc)), yc)
    phi = jnp.arctan2(x, z)
    u = phi / (2.0 * math.pi) + 0.5
    u = u - jnp.floor(u)
    v = theta / math.pi
    xf = jnp.clip(u * W, 0.0, W - 1.0)
    yf = jnp.clip(v * H, 0.0, H - 1.0)
    x0f = jnp.floor(xf)
    y0f = jnp.floor(yf)
    wx_ref[...] = xf - x0f
    wy_ref[...] = yf - y0f
    x0 = x0f.astype(jnp.int32)
    y0 = y0f.astype(jnp.int32)
    x1 = jnp.minimum(x0 + 1, int(W) - 1)
    y1 = jnp.minimum(y0 + 1, int(H) - 1)
    r0 = y0 * int(W)
    r1 = y1 * int(W)
    i00_ref[...] = r0 + x0
    i01_ref[...] = r0 + x1
    i10_ref[...] = r1 + x0
    i11_ref[...] = r1 + x1


def _uv_kernel(ldT, H, W, TB=8192):
    B = ldT.shape[1]
    G = B // TB
    iout = jax.ShapeDtypeStruct((B,), jnp.int32)
    fout = jax.ShapeDtypeStruct((B,), jnp.float32)
    ospec = pl.BlockSpec((TB,), lambda i: (i,))
    outs = pl.pallas_call(
        functools.partial(_uv_body, float(W), float(H)),
        grid=(G,),
        in_specs=[pl.BlockSpec((3, TB), lambda i: (0, i))],
        out_specs=[ospec] * 6,
        out_shape=[iout, iout, iout, iout, fout, fout],
    )(ldT)
    return outs


_LANES = 16


def _sc_gather_combine(env_flat, i00, i01, i10, i11, wx, wy, HW, C=2048):
    B = i00.shape[0]
    info = plsc.get_sparse_core_info()
    NC, NS = info.num_cores, info.num_subcores
    NW = NC * NS
    RW = B // NW
    NCHUNK = RW // C
    GROUPS = C // _LANES
    assert NCHUNK % 2 == 0

    mesh = plsc.VectorSubcoreMesh(core_axis_name="c", subcore_axis_name="s")
    fout = jax.ShapeDtypeStruct((B,), jnp.float32)

    buf_set = [
        pltpu.VMEM((4, C), jnp.int32),
        pltpu.VMEM((8, C), jnp.int32),
        pltpu.VMEM((12, C), jnp.float32),
        pltpu.VMEM((2, C), jnp.float32),
    ]

    @functools.partial(
        pl.kernel,
        out_type=[fout, fout, fout],
        mesh=mesh,
        scratch_types=buf_set + buf_set + [
            pltpu.VMEM((3, C), jnp.float32),
            pltpu.SemaphoreType.DMA,
            pltpu.SemaphoreType.DMA,
        ],
        compiler_params=pltpu.CompilerParams(
            needs_layout_passes=False, use_tc_tiling_on_sc=False),
    )
    def body(env_hbm, i00_hbm, i01_hbm, i10_hbm, i11_hbm, wx_hbm, wy_hbm,
             le0_hbm, le1_hbm, le2_hbm,
             icA, ixA, txA, wA, icB, ixB, txB, wB, out_v, semA, semB):
        wid = lax.axis_index("s") * NC + lax.axis_index("c")
        base = wid * RW
        corners = (i00_hbm, i01_hbm, i10_hbm, i11_hbm)
        sets = ((icA, ixA, txA, wA, semA), (icB, ixB, txB, wB, semB))

        def stage_in(t, p):
            ic, ix, tx, w, sem = sets[p]
            b0 = base + t * C
            for corner in range(4):
                pltpu.sync_copy(corners[corner].at[pl.ds(b0, C)],
                                ic.at[corner])
            pltpu.sync_copy(wx_hbm.at[pl.ds(b0, C)], w.at[0])
            pltpu.sync_copy(wy_hbm.at[pl.ds(b0, C)], w.at[1])

            def build(g, gcarry):
                s = pl.ds(g * _LANES, _LANES)
                for corner in range(4):
                    raw = ic[corner, s]
                    ix[2 * corner, s] = raw + HW
                    ix[2 * corner + 1, s] = raw + 2 * HW
                return gcarry

            lax.fori_loop(0, GROUPS, build, 0)
            for corner in range(4):
                pltpu.async_copy(env_hbm.at[ic.at[corner]],
                                 tx.at[3 * corner], sem)
                pltpu.async_copy(env_hbm.at[ix.at[2 * corner]],
                                 tx.at[3 * corner + 1], sem)
                pltpu.async_copy(env_hbm.at[ix.at[2 * corner + 1]],
                                 tx.at[3 * corner + 2], sem)

        def stage_out(t, p):
            ic, ix, tx, w, sem = sets[p]
            b0 = base + t * C
            for j in range(12):
                pltpu.make_async_copy(env_hbm.at[pl.ds(0, C)],
                                      tx.at[j], sem).wait()

            def group(g, gcarry):
                s = pl.ds(g * _LANES, _LANES)
                wxv = w[0, s]
                wyv = w[1, s]
                for ch in range(3):
                    a = tx[0 + ch, s]
                    b = tx[3 + ch, s]
                    c = tx[6 + ch, s]
                    d = tx[9 + ch, s]
                    top = a + wxv * (b - a)
                    bot = c + wxv * (d - c)
                    out_v[ch, s] = (top + wyv * (bot - top)) * (1.0 / 256.0)
                return gcarry

            lax.fori_loop(0, GROUPS, group, 0)
            pltpu.sync_copy(out_v.at[0], le0_hbm.at[pl.ds(b0, C)])
            pltpu.sync_copy(out_v.at[1], le1_hbm.at[pl.ds(b0, C)])
            pltpu.sync_copy(out_v.at[2], le2_hbm.at[pl.ds(b0, C)])

        stage_in(0, 0)

        def piter(i, carry):
            T = 2 * i
            stage_in(T + 1, 1)
            stage_out(T, 0)

            @pl.when(T + 2 < NCHUNK)
            def _():
                stage_in(T + 2, 0)

            stage_out(T + 1, 1)
            return carry

        lax.fori_loop(0, NCHUNK // 2, piter, 0)

    le0, le1, le2 = body(env_flat, i00, i01, i10, i11, wx, wy)
    return jnp.stack([le0, le1, le2], axis=-1)


def kernel(position, light_dir, envmap):
    B = light_dir.shape[0]
    H, W = envmap.shape[1], envmap.shape[2]
    ldT = light_dir.T
    i00, i01, i10, i11, wx, wy = _uv_kernel(ldT, H, W)
    env_flat = envmap.reshape(3 * H * W)
    le = _sc_gather_combine(env_flat, i00, i01, i10, i11, wx, wy, H * W)
    pdf = jnp.full((B, 1), 1.0 / (4 * math.pi), dtype=jnp.float32)
    valid = jnp.ones((B, 1), dtype=bool)
    return (le, pdf, valid)


# speedup vs baseline: 2.4162x; 1.0219x over previous
import functools
import math

import jax
import jax.numpy as jnp
from jax import lax
from jax.experimental import pallas as pl
from jax.experimental.pallas import tpu as pltpu
from jax.experimental.pallas import tpu_sc as plsc


def _uv_body(W, H, ld_ref, i00_ref, i01_ref, i10_ref, i11_ref, wx_ref, wy_ref):
    x = ld_ref[0, :]
    y = ld_ref[1, :]
    z = ld_ref[2, :]
    norm = jnp.sqrt(x * x + y * y + z * z)
    yn = y / norm
    yc = jnp.clip(yn, -1.0 + 1e-06, 1.0 - 1e-06)
    theta = jnp.arctan2(jnp.sqrt((1.0 + yc) * (1.0 - yc)), yc)
    phi = jnp.arctan2(x, z)
    u = phi / (2.0 * math.pi) + 0.5
    u = u - jnp.floor(u)
    v = theta / math.pi
    xf = jnp.clip(u * W, 0.0, W - 1.0)
    yf = jnp.clip(v * H, 0.0, H - 1.0)
    x0f = jnp.floor(xf)
    y0f = jnp.floor(yf)
    wx_ref[...] = xf - x0f
    wy_ref[...] = yf - y0f
    x0 = x0f.astype(jnp.int32)
    y0 = y0f.astype(jnp.int32)
    x1 = jnp.minimum(x0 + 1, int(W) - 1)
    y1 = jnp.minimum(y0 + 1, int(H) - 1)
    r0 = y0 * int(W)
    r1 = y1 * int(W)
    i00_ref[...] = r0 + x0
    i01_ref[...] = r0 + x1
    i10_ref[...] = r1 + x0
    i11_ref[...] = r1 + x1


def _uv_kernel(ldT, H, W, TB=8192):
    B = ldT.shape[1]
    G = B // TB
    iout = jax.ShapeDtypeStruct((B,), jnp.int32)
    fout = jax.ShapeDtypeStruct((B,), jnp.float32)
    ospec = pl.BlockSpec((TB,), lambda i: (i,))
    outs = pl.pallas_call(
        functools.partial(_uv_body, float(W), float(H)),
        grid=(G,),
        in_specs=[pl.BlockSpec((3, TB), lambda i: (0, i))],
        out_specs=[ospec] * 6,
        out_shape=[iout, iout, iout, iout, fout, fout],
    )(ldT)
    return outs


_LANES = 16


def _sc_gather_combine(env_flat, i00, i01, i10, i11, wx, wy, HW, C=2048):
    B = i00.shape[0]
    info = plsc.get_sparse_core_info()
    NC, NS = info.num_cores, info.num_subcores
    NW = NC * NS
    RW = B // NW
    NCHUNK = RW // C
    GROUPS = C // _LANES
    assert NCHUNK % 2 == 0

    mesh = plsc.VectorSubcoreMesh(core_axis_name="c", subcore_axis_name="s")
    fout = jax.ShapeDtypeStruct((B,), jnp.float32)

    buf_set = [
        pltpu.VMEM((4, C), jnp.int32),
        pltpu.VMEM((8, C), jnp.int32),
        pltpu.VMEM((12, C), jnp.float32),
        pltpu.VMEM((2, C), jnp.float32),
    ]

    @functools.partial(
        pl.kernel,
        out_type=[fout, fout, fout],
        mesh=mesh,
        scratch_types=buf_set + buf_set + [
            pltpu.VMEM((3, C), jnp.float32),
            pltpu.SemaphoreType.DMA,
            pltpu.SemaphoreType.DMA,
            pltpu.SemaphoreType.DMA,
            pltpu.SemaphoreType.DMA,
            pltpu.SemaphoreType.DMA,
            pltpu.SemaphoreType.DMA,
        ],
        compiler_params=pltpu.CompilerParams(
            needs_layout_passes=False, use_tc_tiling_on_sc=False),
    )
    def body(env_hbm, i00_hbm, i01_hbm, i10_hbm, i11_hbm, wx_hbm, wy_hbm,
             le0_hbm, le1_hbm, le2_hbm,
             icA, ixA, txA, wA, icB, ixB, txB, wB, out_v,
             semA, semB, sinA, sinB, swA, swB):
        wid = lax.axis_index("s") * NC + lax.axis_index("c")
        base = wid * RW
        corners = (i00_hbm, i01_hbm, i10_hbm, i11_hbm)
        sets = ((icA, ixA, txA, wA, semA, sinA, swA),
                (icB, ixB, txB, wB, semB, sinB, swB))

        def fire_ic(t, p):
            ic, ix, tx, w, sem, sin, sw = sets[p]
            b0 = base + t * C
            for corner in range(4):
                pltpu.async_copy(corners[corner].at[pl.ds(b0, C)],
                                 ic.at[corner], sin)

        def fire_w(t, p):
            ic, ix, tx, w, sem, sin, sw = sets[p]
            b0 = base + t * C
            pltpu.async_copy(wx_hbm.at[pl.ds(b0, C)], w.at[0], sw)
            pltpu.async_copy(wy_hbm.at[pl.ds(b0, C)], w.at[1], sw)

        def buildfire(p):
            ic, ix, tx, w, sem, sin, sw = sets[p]
            for corner in range(4):
                pltpu.make_async_copy(i00_hbm.at[pl.ds(0, C)],
                                      ic.at[corner], sin).wait()

            def build(g, gcarry):
                s = pl.ds(g * _LANES, _LANES)
                for corner in range(4):
                    raw = ic[corner, s]
                    ix[2 * corner, s] = raw + HW
                    ix[2 * corner + 1, s] = raw + 2 * HW
                return gcarry

            lax.fori_loop(0, GROUPS, build, 0)
            for corner in range(4):
                pltpu.async_copy(env_hbm.at[ic.at[corner]],
                                 tx.at[3 * corner], sem)
                pltpu.async_copy(env_hbm.at[ix.at[2 * corner]],
                                 tx.at[3 * corner + 1], sem)
                pltpu.async_copy(env_hbm.at[ix.at[2 * corner + 1]],
                                 tx.at[3 * corner + 2], sem)

        def drain(p):
            ic, ix, tx, w, sem, sin, sw = sets[p]
            for j in range(12):
                pltpu.make_async_copy(env_hbm.at[pl.ds(0, C)],
                                      tx.at[j], sem).wait()

        def combine_out(t, p):
            ic, ix, tx, w, sem, sin, sw = sets[p]
            b0 = base + t * C
            for j in range(2):
                pltpu.make_async_copy(wx_hbm.at[pl.ds(0, C)],
                                      w.at[j], sw).wait()

            def group(g, gcarry):
                s = pl.ds(g * _LANES, _LANES)
                wxv = w[0, s]
                wyv = w[1, s]
                for ch in range(3):
                    a = tx[0 + ch, s]
                    b = tx[3 + ch, s]
                    c = tx[6 + ch, s]
                    d = tx[9 + ch, s]
                    top = a + wxv * (b - a)
                    bot = c + wxv * (d - c)
                    out_v[ch, s] = (top + wyv * (bot - top)) * (1.0 / 256.0)
                return gcarry

            lax.fori_loop(0, GROUPS, group, 0)
            pltpu.sync_copy(out_v.at[0], le0_hbm.at[pl.ds(b0, C)])
            pltpu.sync_copy(out_v.at[1], le1_hbm.at[pl.ds(b0, C)])
            pltpu.sync_copy(out_v.at[2], le2_hbm.at[pl.ds(b0, C)])

        fire_ic(0, 0)
        fire_w(0, 0)
        fire_ic(1, 1)
        fire_w(1, 1)
        buildfire(0)
        buildfire(1)

        def piter(i, carry):
            T = 2 * i
            drain(0)

            @pl.when(T + 2 < NCHUNK)
            def _():
                fire_ic(T + 2, 0)

            combine_out(T, 0)

            @pl.when(T + 2 < NCHUNK)
            def _():
                fire_w(T + 2, 0)
                buildfire(0)

            drain(1)

            @pl.when(T + 3 < NCHUNK)
            def _():
                fire_ic(T + 3, 1)

            combine_out(T + 1, 1)

            @pl.when(T + 3 < NCHUNK)
            def _():
                fire_w(T + 3, 1)
                buildfire(1)
            return carry

        lax.fori_loop(0, NCHUNK // 2, piter, 0)

    le0, le1, le2 = body(env_flat, i00, i01, i10, i11, wx, wy)
    return jnp.stack([le0, le1, le2], axis=-1)


def kernel(position, light_dir, envmap):
    B = light_dir.shape[0]
    H, W = envmap.shape[1], envmap.shape[2]
    ldT = light_dir.T
    i00, i01, i10, i11, wx, wy = _uv_kernel(ldT, H, W)
    env_flat = envmap.reshape(3 * H * W)
    le = _sc_gather_combine(env_flat, i00, i01, i10, i11, wx, wy, H * W)
    pdf = jnp.full((B, 1), 1.0 / (4 * math.pi), dtype=jnp.float32)
    valid = jnp.ones((B, 1), dtype=bool)
    return (le, pdf, valid)
